# baseline (speedup 1.0000x reference)
import jax
import jax.numpy as jnp
from jax.experimental import pallas as pl
from jax.experimental.pallas import tpu as pltpu

_NUM_EXPERTS = 64
_TILE = 2048


def _gate_half(x_ref, w_ref, b_ref, sparse_ref, idx_ref, topw_ref):
    t = x_ref.shape[0]
    e = _NUM_EXPERTS
    logits = jax.lax.dot_general(
        x_ref[...], w_ref[...],
        dimension_numbers=(((1,), (1,)), ((), ())),
        preferred_element_type=jnp.float32,
    ) + b_ref[...]

    iota = jax.lax.broadcasted_iota(jnp.int32, (t, e), 1).astype(jnp.float32)
    fill = jnp.float32(e)
    m0 = jnp.max(logits, axis=1, keepdims=True)
    i0 = jnp.min(jnp.where(logits == m0, iota, fill), axis=1, keepdims=True)
    sel0 = iota == i0
    masked = jnp.where(sel0, -jnp.inf, logits)
    m1 = jnp.max(masked, axis=1, keepdims=True)
    i1 = jnp.min(jnp.where(masked == m1, iota, fill), axis=1, keepdims=True)
    sel1 = iota == i1

    z = jnp.exp(m1 - m0)
    w0 = 1.0 / (1.0 + z)
    w1 = z / (1.0 + z)

    sparse_ref[...] = jnp.where(sel0, w0, 0.0) + jnp.where(sel1, w1, 0.0)
    idx_ref[...] = jnp.concatenate([i0, i1], axis=1).astype(jnp.int32)
    topw_ref[...] = jnp.concatenate([w0, w1], axis=1)


def _gate_body(xa_ref, xb_ref, w_ref, b_ref,
               sa_ref, ia_ref, ta_ref, sb_ref, ib_ref, tb_ref):
    _gate_half(xa_ref, w_ref, b_ref, sa_ref, ia_ref, ta_ref)
    _gate_half(xb_ref, w_ref, b_ref, sb_ref, ib_ref, tb_ref)


def kernel(x, W, b):
    n, d = x.shape
    e = _NUM_EXPERTS
    h = n // 2
    grid = h // _TILE
    b2 = b.reshape(1, e)
    sa, ia, ta, sb, ib, tb = pl.pallas_call(
        _gate_body,
        grid=(grid,),
        in_specs=[
            pl.BlockSpec((_TILE, d), lambda i: (i, 0)),
            pl.BlockSpec((_TILE, d), lambda i: (i + grid, 0)),
            pl.BlockSpec((e, d), lambda i: (0, 0)),
            pl.BlockSpec((1, e), lambda i: (0, 0)),
        ],
        out_specs=[
            pl.BlockSpec((_TILE, e), lambda i: (i, 0)),
            pl.BlockSpec((_TILE, 2), lambda i: (i, 0)),
            pl.BlockSpec((_TILE, 2), lambda i: (i, 0)),
            pl.BlockSpec((_TILE, e), lambda i: (i, 0)),
            pl.BlockSpec((_TILE, 2), lambda i: (i, 0)),
            pl.BlockSpec((_TILE, 2), lambda i: (i, 0)),
        ],
        out_shape=[
            jax.ShapeDtypeStruct((h, e), x.dtype),
            jax.ShapeDtypeStruct((h, 2), jnp.int32),
            jax.ShapeDtypeStruct((h, 2), jnp.float32),
            jax.ShapeDtypeStruct((h, e), x.dtype),
            jax.ShapeDtypeStruct((h, 2), jnp.int32),
            jax.ShapeDtypeStruct((h, 2), jnp.float32),
        ],
        compiler_params=pltpu.CompilerParams(
            vmem_limit_bytes=63 * 1024 * 1024,
        ),
    )(x, x, W, b2)
    sparse = jnp.concatenate([sa, sb], axis=0)
    idx = jnp.concatenate([ia, ib], axis=0)
    topw = jnp.concatenate([ta, tb], axis=0)
    return (sparse, idx, topw)


# speedup vs baseline: 1.2956x; 1.2956x over previous
import jax
import jax.numpy as jnp
from jax.experimental import pallas as pl
from jax.experimental.pallas import tpu as pltpu

_NUM_EXPERTS = 64
_TILE = 4096


def _gate_body(x_ref, w_ref, b_ref, sparse_ref, idx_ref, topw_ref):
    t = x_ref.shape[0]
    e = _NUM_EXPERTS
    logits = jax.lax.dot_general(
        x_ref[...], w_ref[...],
        dimension_numbers=(((1,), (1,)), ((), ())),
        preferred_element_type=jnp.float32,
    ) + b_ref[...]

    iota = jax.lax.broadcasted_iota(jnp.int32, (t, e), 1).astype(jnp.float32)
    fill = jnp.float32(e)
    m0 = jnp.max(logits, axis=1, keepdims=True)
    i0 = jnp.min(jnp.where(logits == m0, iota, fill), axis=1, keepdims=True)
    sel0 = iota == i0
    masked = jnp.where(sel0, -jnp.inf, logits)
    m1 = jnp.max(masked, axis=1, keepdims=True)
    i1 = jnp.min(jnp.where(masked == m1, iota, fill), axis=1, keepdims=True)
    sel1 = iota == i1

    z = jnp.exp(m1 - m0)
    w0 = 1.0 / (1.0 + z)
    w1 = z / (1.0 + z)

    sparse_ref[...] = jnp.where(sel0, w0, 0.0) + jnp.where(sel1, w1, 0.0)
    idx_ref[...] = jnp.concatenate([i0, i1], axis=1).astype(jnp.int32)
    topw_ref[...] = jnp.concatenate([w0, w1], axis=1)


def kernel(x, W, b):
    n, d = x.shape
    e = _NUM_EXPERTS
    grid = n // _TILE
    b2 = b.reshape(1, e)
    sparse, idx, topw = pl.pallas_call(
        _gate_body,
        grid=(grid,),
        in_specs=[
            pl.BlockSpec((_TILE, d), lambda i: (i, 0)),
            pl.BlockSpec((e, d), lambda i: (0, 0)),
            pl.BlockSpec((1, e), lambda i: (0, 0)),
        ],
        out_specs=[
            pl.BlockSpec((_TILE, e), lambda i: (i, 0)),
            pl.BlockSpec((_TILE, 2), lambda i: (i, 0)),
            pl.BlockSpec((_TILE, 2), lambda i: (i, 0)),
        ],
        out_shape=[
            jax.ShapeDtypeStruct((n, e), x.dtype),
            jax.ShapeDtypeStruct((n, 2), jnp.int32),
            jax.ShapeDtypeStruct((n, 2), jnp.float32),
        ],
    )(x, W, b2)
    return (sparse, idx, topw)
